# Initial kernel scaffold; baseline (speedup 1.0000x reference)
import jax
import jax.numpy as jnp
from jax.experimental import pallas as pl


def kernel(pos, beat_table, bar_table):
    raise NotImplementedError("write your pallas kernel here")


# speedup vs baseline: 8.1610x; 8.1610x over previous
import functools

import jax
import jax.numpy as jnp
from jax import lax
from jax.experimental import pallas as pl
from jax.experimental.pallas import tpu as pltpu
from jax.experimental.pallas import tpu_sc as plsc

BEAT = 32
BARS = 1024
EMB = 64
NROWS = BEAT * BARS


def _build_combined(beat_ref, bar_ref, out_ref):
    out_ref[...] = bar_ref[...][:, None, :] + beat_ref[...][None, :, :]


def _make_sc_gather(total_rows, chunk_rows, sub_rows):
    info = plsc.get_sparse_core_info()
    nw = info.num_cores * info.num_subcores
    assert total_rows % (nw * chunk_rows) == 0
    rows_per_w = total_rows // nw
    n_chunks = rows_per_w // chunk_rows
    n_sub = chunk_rows // sub_rows
    mesh = plsc.VectorSubcoreMesh(core_axis_name="c", subcore_axis_name="s")

    @functools.partial(
        pl.kernel,
        out_type=jax.ShapeDtypeStruct((total_rows, EMB), jnp.float32),
        mesh=mesh,
        scratch_types=[
            pltpu.VMEM((n_sub, sub_rows), jnp.int32),
            pltpu.VMEM((chunk_rows, EMB), jnp.float32),
            pltpu.SemaphoreType.DMA,
        ],
        compiler_params=pltpu.CompilerParams(use_tc_tiling_on_sc=False),
    )
    def gather_kernel(comb_hbm, idx_hbm, out_hbm, idx_v, rows_v, sem):
        wid = lax.axis_index("s") * info.num_cores + lax.axis_index("c")
        base = wid * rows_per_w

        def chunk_body(i, carry):
            off = pl.multiple_of(base + i * chunk_rows, chunk_rows)
            idx_off = pl.multiple_of(off // sub_rows, n_sub)
            pltpu.sync_copy(idx_hbm.at[pl.ds(idx_off, n_sub)], idx_v)
            handles = [
                pltpu.async_copy(
                    comb_hbm.at[idx_v.at[j]],
                    rows_v.at[pl.ds(j * sub_rows, sub_rows)],
                    sem,
                )
                for j in range(n_sub)
            ]
            for h in handles:
                h.wait()
            pltpu.sync_copy(rows_v, out_hbm.at[pl.ds(off, chunk_rows)])
            return carry

        lax.fori_loop(0, n_chunks, chunk_body, 0)

    return gather_kernel


def kernel(pos, beat_table, bar_table):
    b, t = pos.shape
    total = b * t

    comb3 = pl.pallas_call(
        _build_combined,
        out_shape=jax.ShapeDtypeStruct((BARS, BEAT, EMB), jnp.float32),
    )(beat_table, bar_table)
    comb = comb3.reshape(NROWS, EMB)

    sub_rows = 128
    chunk_rows = 1024
    idx2d = pos.reshape(total // sub_rows, sub_rows)
    gather = _make_sc_gather(total, chunk_rows, sub_rows)
    out = gather(comb, idx2d)
    return out.reshape(b, t, EMB)


# speedup vs baseline: 8.3578x; 1.0241x over previous
import functools

import jax
import jax.numpy as jnp
from jax import lax
from jax.experimental import pallas as pl
from jax.experimental.pallas import tpu as pltpu
from jax.experimental.pallas import tpu_sc as plsc

BEAT = 32
BARS = 1024
EMB = 64
NROWS = BEAT * BARS


def _build_combined(beat_ref, bar_ref, out_ref):
    out_ref[...] = bar_ref[...][:, None, :] + beat_ref[...][None, :, :]


def _make_sc_gather(total_rows, chunk_rows, sub_rows):
    info = plsc.get_sparse_core_info()
    nw = info.num_cores * info.num_subcores
    assert total_rows % (nw * 2 * chunk_rows) == 0
    rows_per_w = total_rows // nw
    n_steps = rows_per_w // (2 * chunk_rows)
    n_sub = chunk_rows // sub_rows
    mesh = plsc.VectorSubcoreMesh(core_axis_name="c", subcore_axis_name="s")

    @functools.partial(
        pl.kernel,
        out_type=jax.ShapeDtypeStruct((total_rows, EMB), jnp.float32),
        mesh=mesh,
        scratch_types=[
            pltpu.VMEM((2 * n_sub, sub_rows), jnp.int32),
            pltpu.VMEM((2 * chunk_rows, EMB), jnp.float32),
            pltpu.SemaphoreType.DMA,
            pltpu.SemaphoreType.DMA,
            pltpu.SemaphoreType.DMA,
            pltpu.SemaphoreType.DMA,
        ],
        compiler_params=pltpu.CompilerParams(use_tc_tiling_on_sc=False),
    )
    def gather_kernel(comb_hbm, idx_hbm, out_hbm, idx_v, rows_v, g0, g1, o0, o1):
        sems_g = (g0, g1)
        sems_o = (o0, o1)
        wid = lax.axis_index("s") * info.num_cores + lax.axis_index("c")
        base = wid * rows_per_w

        def rows_buf(b):
            return rows_v.at[pl.ds(b * chunk_rows, chunk_rows)]

        def step(i, carry):
            off0 = pl.multiple_of(base + i * 2 * chunk_rows, 2 * chunk_rows)
            idx_off = pl.multiple_of(off0 // sub_rows, 2 * n_sub)
            pltpu.sync_copy(idx_hbm.at[pl.ds(idx_off, 2 * n_sub)], idx_v)
            handles = []
            for b in range(2):
                @pl.when(i > 0)
                def _(b=b):
                    pltpu.make_async_copy(
                        rows_buf(b),
                        out_hbm.at[pl.ds(off0, chunk_rows)],
                        sems_o[b],
                    ).wait()

                handles.append([
                    pltpu.async_copy(
                        comb_hbm.at[idx_v.at[b * n_sub + j]],
                        rows_v.at[pl.ds(b * chunk_rows + j * sub_rows, sub_rows)],
                        sems_g[b],
                    )
                    for j in range(n_sub)
                ])
            for b in range(2):
                for h in handles[b]:
                    h.wait()
                off = pl.multiple_of(off0 + b * chunk_rows, chunk_rows)
                pltpu.async_copy(
                    rows_buf(b), out_hbm.at[pl.ds(off, chunk_rows)], sems_o[b]
                )
            return carry

        lax.fori_loop(0, n_steps, step, 0)
        for b in range(2):
            pltpu.make_async_copy(
                rows_buf(b),
                out_hbm.at[pl.ds(base, chunk_rows)],
                sems_o[b],
            ).wait()

    return gather_kernel


def kernel(pos, beat_table, bar_table):
    b, t = pos.shape
    total = b * t

    comb3 = pl.pallas_call(
        _build_combined,
        out_shape=jax.ShapeDtypeStruct((BARS, BEAT, EMB), jnp.float32),
    )(beat_table, bar_table)
    comb = comb3.reshape(NROWS, EMB)

    sub_rows = 128
    chunk_rows = 512
    idx2d = pos.reshape(total // sub_rows, sub_rows)
    gather = _make_sc_gather(total, chunk_rows, sub_rows)
    out = gather(comb, idx2d)
    return out.reshape(b, t, EMB)
